# Initial kernel scaffold; baseline (speedup 1.0000x reference)
import jax
import jax.numpy as jnp
from jax.experimental import pallas as pl


def kernel(latents, codebooks):
    raise NotImplementedError("write your pallas kernel here")


# speedup vs baseline: 4.2061x; 4.2061x over previous
import functools

import jax
import jax.numpy as jnp
from jax import lax
from jax.experimental import pallas as pl
from jax.experimental.pallas import tpu as pltpu
from jax.experimental.pallas import tpu_sc as plsc

G = 8
K = 1024
D = 64
N = 9216
BN = 512
NBLK = N // BN

NC = 2
NS = 16
NW = NC * NS
B = N * G
B_PER_W = B // NW
CHUNK = 576
NCHUNK = B_PER_W // CHUNK


def _tc_body(x_ref, cb_ref, idx_ref, loss_ref):
    acc = jnp.float32(0.0)
    for g in range(G):
        xg = x_ref[:, g * D:(g + 1) * D]
        cbg = cb_ref[g]
        xw = lax.dot_general(xg, cbg, (((1,), (1,)), ((), ())),
                             preferred_element_type=jnp.float32)
        x2 = jnp.sum(xg * xg, axis=-1, keepdims=True)
        w2 = jnp.sum(cbg * cbg, axis=-1)
        dist = (x2 - 2.0 * xw) + w2[None, :]
        m = jnp.min(dist, axis=-1, keepdims=True)
        iota = lax.broadcasted_iota(jnp.int32, dist.shape, 1)
        idx = jnp.min(jnp.where(dist == m, iota, K), axis=-1)
        idx_ref[g, :] = idx + g * K
        acc = acc + jnp.sum(m)
    loss_ref[...] = acc.reshape(1, 1, 1)


def _tc_distance_argmin(flat, codebooks):
    return pl.pallas_call(
        _tc_body,
        grid=(NBLK,),
        in_specs=[
            pl.BlockSpec((BN, G * D), lambda i: (i, 0)),
            pl.BlockSpec((G, K, D), lambda i: (0, 0, 0)),
        ],
        out_specs=[
            pl.BlockSpec((G, BN), lambda i: (0, i)),
            pl.BlockSpec((1, 1, 1), lambda i: (i, 0, 0)),
        ],
        out_shape=[
            jax.ShapeDtypeStruct((G, N), jnp.int32),
            jax.ShapeDtypeStruct((NBLK, 1, 1), jnp.float32),
        ],
    )(flat, codebooks)


def _sc_decode(table, idx_flat):
    mesh = plsc.VectorSubcoreMesh(core_axis_name="c", subcore_axis_name="s")

    @functools.partial(
        pl.kernel, mesh=mesh,
        out_type=jax.ShapeDtypeStruct((B, 2 * D), jnp.float32),
        scratch_types=[
            pltpu.VMEM((CHUNK,), jnp.int32),
            pltpu.VMEM((CHUNK, 2 * D), jnp.float32),
            pltpu.SemaphoreType.DMA,
        ],
    )
    def k(table_hbm, idx_hbm, out_hbm, idx_v, rows_v, sem):
        wid = lax.axis_index("s") * NC + lax.axis_index("c")
        base = wid * B_PER_W
        for c in range(NCHUNK):
            off = base + c * CHUNK
            pltpu.sync_copy(idx_hbm.at[pl.ds(off, CHUNK)], idx_v)
            pltpu.async_copy(table_hbm.at[idx_v], rows_v, sem).wait()
            pltpu.sync_copy(rows_v, out_hbm.at[pl.ds(off, CHUNK)])

    return k(table, idx_flat)


def kernel(latents, codebooks):
    flat = latents.reshape(N, G * D)
    idx_all, loss_parts = _tc_distance_argmin(flat, codebooks)
    idx_flat = idx_all.T.reshape(B)
    cb_flat = codebooks.reshape(G * K, D)
    table = jnp.concatenate([cb_flat, cb_flat], axis=1)
    quant = _sc_decode(table, idx_flat)
    quantized = quant[:, :D].reshape(latents.shape)
    vq_loss = jnp.sum(loss_parts) * (1.25 / (N * G * D))
    return (quantized, vq_loss)


# speedup vs baseline: 5.0899x; 1.2101x over previous
import functools

import jax
import jax.numpy as jnp
from jax import lax
from jax.experimental import pallas as pl
from jax.experimental.pallas import tpu as pltpu
from jax.experimental.pallas import tpu_sc as plsc

G = 8
K = 1024
D = 64
N = 9216
BN = 512
NBLK = N // BN

NC = 2
NS = 16
NW = NC * NS
B = N * G
B_PER_W = B // NW
CHUNK = 576
NCHUNK = B_PER_W // CHUNK


def _tc_body(x_ref, cb_ref, idx_ref, loss_ref, w2_ref, cbn2_ref, iota_ref):
    @pl.when(pl.program_id(0) == 0)
    def _init():
        for g in range(G):
            cbg = cb_ref[g]
            cbn2_ref[g] = -2.0 * cbg
            w2_ref[g] = jnp.sum(cbg * cbg, axis=-1).reshape(1, K)
        iota_ref[...] = lax.broadcasted_iota(
            jnp.int32, (8, K), 1).astype(jnp.float32)

    acc = jnp.float32(0.0)
    for g in range(G):
        xg = x_ref[:, g * D:(g + 1) * D]
        xw2 = lax.dot_general(xg, cbn2_ref[g], (((1,), (1,)), ((), ())),
                              preferred_element_type=jnp.float32)
        x2 = jnp.sum(xg * xg, axis=-1, keepdims=True)
        dist = (x2 + xw2) + w2_ref[g]
        m = jnp.min(dist, axis=-1, keepdims=True)
        iota = iota_ref[0:1, :]
        idxf = jnp.min(jnp.where(dist == m, iota, jnp.float32(K)), axis=-1)
        idx_ref[g, :] = idxf.astype(jnp.int32) + g * K
        acc = acc + jnp.sum(m)
    loss_ref[...] = acc.reshape(1, 1, 1)


def _tc_distance_argmin(flat, codebooks):
    return pl.pallas_call(
        _tc_body,
        grid=(NBLK,),
        in_specs=[
            pl.BlockSpec((BN, G * D), lambda i: (i, 0)),
            pl.BlockSpec((G, K, D), lambda i: (0, 0, 0)),
        ],
        out_specs=[
            pl.BlockSpec((G, BN), lambda i: (0, i)),
            pl.BlockSpec((1, 1, 1), lambda i: (i, 0, 0)),
        ],
        out_shape=[
            jax.ShapeDtypeStruct((G, N), jnp.int32),
            jax.ShapeDtypeStruct((NBLK, 1, 1), jnp.float32),
        ],
        scratch_shapes=[
            pltpu.VMEM((G, 1, K), jnp.float32),
            pltpu.VMEM((G, K, D), jnp.float32),
            pltpu.VMEM((8, K), jnp.float32),
        ],
    )(flat, codebooks)


def _sc_decode(table, idx_flat):
    mesh = plsc.VectorSubcoreMesh(core_axis_name="c", subcore_axis_name="s")

    @functools.partial(
        pl.kernel, mesh=mesh,
        out_type=jax.ShapeDtypeStruct((B, 2 * D), jnp.float32),
        scratch_types=[
            pltpu.VMEM((CHUNK,), jnp.int32),
            pltpu.VMEM((CHUNK, 2 * D), jnp.float32),
            pltpu.SemaphoreType.DMA,
        ],
    )
    def k(table_hbm, idx_hbm, out_hbm, idx_v, rows_v, sem):
        wid = lax.axis_index("s") * NC + lax.axis_index("c")
        base = wid * B_PER_W
        for c in range(NCHUNK):
            off = base + c * CHUNK
            pltpu.sync_copy(idx_hbm.at[pl.ds(off, CHUNK)], idx_v)
            pltpu.async_copy(table_hbm.at[idx_v], rows_v, sem).wait()
            pltpu.sync_copy(rows_v, out_hbm.at[pl.ds(off, CHUNK)])

    return k(table, idx_flat)


def kernel(latents, codebooks):
    flat = latents.reshape(N, G * D)
    idx_all, loss_parts = _tc_distance_argmin(flat, codebooks)
    cb_flat = codebooks.reshape(G * K, D)
    table = jnp.concatenate([cb_flat, cb_flat], axis=1)
    quant = _sc_decode(table, idx_all.T.reshape(B))
    quantized = quant[:, :D].reshape(latents.shape)
    vq_loss = jnp.sum(loss_parts) * (1.25 / (N * G * D))
    return (quantized, vq_loss)


# speedup vs baseline: 5.9067x; 1.1605x over previous
import functools

import jax
import jax.numpy as jnp
from jax import lax
from jax.experimental import pallas as pl
from jax.experimental.pallas import tpu as pltpu
from jax.experimental.pallas import tpu_sc as plsc

G = 8
K = 1024
D = 64
N = 9216
BN = 512
NBLK = N // BN

NC = 2
NS = 16
NW = NC * NS
B = N * G
B_PER_W = B // NW
CHUNK = 576
NCHUNK = B_PER_W // CHUNK


def _tc_body(x_ref, cb_ref, idx_ref, loss_ref, w2_ref, cbn2_ref, iota_ref):
    @pl.when(pl.program_id(0) == 0)
    def _init():
        for g in range(G):
            cbg = cb_ref[g]
            cbn2_ref[g] = -2.0 * cbg
            w2_ref[g] = jnp.sum(cbg * cbg, axis=-1).reshape(1, K)
        iota_ref[...] = lax.broadcasted_iota(
            jnp.int32, (8, K), 1).astype(jnp.float32)

    acc = jnp.float32(0.0)
    for g in range(G):
        xg = x_ref[:, g * D:(g + 1) * D]
        xw2 = lax.dot_general(xg, cbn2_ref[g], (((1,), (1,)), ((), ())),
                              preferred_element_type=jnp.float32)
        x2 = jnp.sum(xg * xg, axis=-1, keepdims=True)
        dist = (x2 + xw2) + w2_ref[g]
        m = jnp.min(dist, axis=-1, keepdims=True)
        iota = iota_ref[0:1, :]
        idxf = jnp.min(jnp.where(dist == m, iota, jnp.float32(K)),
                       axis=-1, keepdims=True)
        idx_ref[:, g:g + 1] = idxf.astype(jnp.int32) + g * K
        acc = acc + jnp.sum(m)
    loss_ref[...] = acc.reshape(1, 1, 1)


def _tc_distance_argmin(flat, codebooks):
    return pl.pallas_call(
        _tc_body,
        grid=(NBLK,),
        in_specs=[
            pl.BlockSpec((BN, G * D), lambda i: (i, 0)),
            pl.BlockSpec((G, K, D), lambda i: (0, 0, 0)),
        ],
        out_specs=[
            pl.BlockSpec((BN, G), lambda i: (i, 0)),
            pl.BlockSpec((1, 1, 1), lambda i: (i, 0, 0)),
        ],
        out_shape=[
            jax.ShapeDtypeStruct((N, G), jnp.int32),
            jax.ShapeDtypeStruct((NBLK, 1, 1), jnp.float32),
        ],
        scratch_shapes=[
            pltpu.VMEM((G, 1, K), jnp.float32),
            pltpu.VMEM((G, K, D), jnp.float32),
            pltpu.VMEM((8, K), jnp.float32),
        ],
    )(flat, codebooks)


def _sc_decode(table, idx_flat):
    mesh = plsc.VectorSubcoreMesh(core_axis_name="c", subcore_axis_name="s")

    @functools.partial(
        pl.kernel, mesh=mesh,
        out_type=jax.ShapeDtypeStruct((B, 2 * D), jnp.float32),
        scratch_types=[
            pltpu.VMEM((CHUNK,), jnp.int32),
            pltpu.VMEM((CHUNK, 2 * D), jnp.float32),
            pltpu.SemaphoreType.DMA,
        ],
    )
    def k(table_hbm, idx_hbm, out_hbm, idx_v, rows_v, sem):
        wid = lax.axis_index("s") * NC + lax.axis_index("c")
        base = wid * B_PER_W
        for c in range(NCHUNK):
            off = base + c * CHUNK
            pltpu.sync_copy(idx_hbm.at[pl.ds(off, CHUNK)], idx_v)
            pltpu.async_copy(table_hbm.at[idx_v], rows_v, sem).wait()
            pltpu.sync_copy(rows_v, out_hbm.at[pl.ds(off, CHUNK)])

    return k(table, idx_flat)


def kernel(latents, codebooks):
    flat = latents.reshape(N, G * D)
    idx_all, loss_parts = _tc_distance_argmin(flat, codebooks)
    cb_flat = codebooks.reshape(G * K, D)
    table = jnp.concatenate([cb_flat, cb_flat], axis=1)
    quant = _sc_decode(table, idx_all.reshape(B))
    quantized = quant[:, :D].reshape(latents.shape)
    vq_loss = jnp.sum(loss_parts) * (1.25 / (N * G * D))
    return (quantized, vq_loss)
